# baseline (device time: 41373 ns/iter reference)
import jax
import jax.numpy as jnp
from jax import lax
from jax.experimental import pallas as pl
from jax.experimental.pallas import tpu as pltpu

N_DEV = 8


def kernel(x, router_W, route_idx, expert_W, shared_W):
    n_tok, d_model = x.shape
    e_per, _, d_ff = expert_W.shape

    def body(x_ref, router_ref, ridx_ref, expW_ref, shared_ref, out_ref,
             comm_ref, send_sems, recv_sems):
        my = lax.axis_index("i")
        left = lax.rem(my + N_DEV - 1, N_DEV)
        right = lax.rem(my + 1, N_DEV)

        barrier_sem = pltpu.get_barrier_semaphore()
        for nbr in (left, right):
            pl.semaphore_signal(
                barrier_sem, inc=1,
                device_id=(nbr,), device_id_type=pl.DeviceIdType.MESH,
            )
        pl.semaphore_wait(barrier_sem, 2)

        xv = x_ref[...]
        scores = jnp.dot(xv, router_ref[...],
                         preferred_element_type=jnp.float32)
        s_max = jnp.max(scores, axis=-1, keepdims=True)
        p = jnp.exp(scores - s_max)
        probs = p / jnp.sum(p, axis=-1, keepdims=True)
        ri = ridx_ref[...]
        e_ids = lax.broadcasted_iota(jnp.int32, scores.shape, 1)
        sel = jnp.sum(jnp.where(ri == e_ids, probs, 0.0),
                      axis=-1, keepdims=True)

        acc = jnp.dot(xv, shared_ref[...],
                      preferred_element_type=jnp.float32)

        def contrib(chunk, origin):
            out = jnp.zeros((n_tok, d_ff), jnp.float32)
            for k in range(e_per):
                eid = e_per * origin + k
                coeff = jnp.where(ri == eid, sel, 0.0)
                out += jnp.dot(xv * coeff, chunk[k],
                               preferred_element_type=jnp.float32)
            return out

        comm_ref[0] = expW_ref[...]
        acc += contrib(comm_ref[0], my)

        for h in range(N_DEV - 1):
            rdma = pltpu.make_async_remote_copy(
                src_ref=comm_ref.at[h],
                dst_ref=comm_ref.at[h + 1],
                send_sem=send_sems.at[h],
                recv_sem=recv_sems.at[h],
                device_id=(right,),
                device_id_type=pl.DeviceIdType.MESH,
            )
            rdma.start()
            rdma.wait()
            origin = lax.rem(my + N_DEV - 1 - h, N_DEV)
            acc += contrib(comm_ref[h + 1], origin)

        out_ref[...] = acc

    return pl.pallas_call(
        body,
        out_shape=jax.ShapeDtypeStruct((n_tok, d_ff), jnp.float32),
        in_specs=[pl.BlockSpec(memory_space=pltpu.VMEM)] * 5,
        out_specs=pl.BlockSpec(memory_space=pltpu.VMEM),
        scratch_shapes=[
            pltpu.VMEM((N_DEV, e_per, d_model, d_ff), jnp.float32),
            pltpu.SemaphoreType.DMA((N_DEV - 1,)),
            pltpu.SemaphoreType.DMA((N_DEV - 1,)),
        ],
        compiler_params=pltpu.CompilerParams(collective_id=0),
    )(x, router_W, route_idx, expert_W, shared_W)
